# baseline (speedup 1.0000x reference)
import functools

import jax
import jax.numpy as jnp
from jax.experimental import pallas as pl
from jax.experimental.pallas import tpu as pltpu

_BATCH = 64
_THRESH2 = 3.0
_ALPHA = 2.0
_BETA = 1.0
_OBS_R = 256
_OBS_S = 512


def _score_body(z_ref, r_ref, out_ref):
    j = pl.program_id(1)
    x = z_ref[0]
    x0 = x[0]
    x1 = x[1]
    x2 = x[2]
    z2 = x0 * x0 + x1 * x1 + x2 * x2
    b_coef = -0.5 / (r_ref[0, 0, 0] * r_ref[0, 0, 0])
    e = jnp.exp(z2 * b_coef)
    scores = jnp.where(z2 < _THRESH2, e, 0.0)
    ssum = jnp.sum(scores)

    @pl.when(j == 0)
    def _init():
        out_ref[0, 0, 0] = ssum

    @pl.when(j != 0)
    def _acc():
        out_ref[0, 0, 0] += ssum


def _finish_body(raw_ref, r_ref, nobs_ref, mu_ref, s2_ref, obj_ref):
    r = r_ref[...]
    a = 1.0 / (r * r)
    b = 0.5 * a
    t2 = _THRESH2
    mu = (1.0 - jnp.exp(-b * t2)) / (4.0 * b)
    mean_s2 = (1.0 - jnp.exp(-2.0 * b * t2)) / (8.0 * b)
    sigma2 = mean_s2 - mu * mu
    n = nobs_ref[0, 0]
    mu = n * mu
    sigma2 = n * sigma2
    mu_ref[...] = mu
    s2_ref[...] = sigma2
    obj_ref[...] = raw_ref[...] - _ALPHA * mu - _BETA * sigma2


@functools.partial(jax.jit, static_argnames=())
def kernel(z, R, num_obs):
    zt = jnp.transpose(z, (0, 3, 1, 2))

    rb = 64
    nj = _OBS_R // rb
    raw2 = pl.pallas_call(
        _score_body,
        grid=(_BATCH, nj),
        in_specs=[
            pl.BlockSpec((1, 3, rb, _OBS_S), lambda b, j: (b, 0, j, 0)),
            pl.BlockSpec((1, 1, 1), lambda b, j: (b, 0, 0),
                         memory_space=pltpu.SMEM),
        ],
        out_specs=pl.BlockSpec((1, 1, 1), lambda b, j: (b, 0, 0),
                               memory_space=pltpu.SMEM),
        out_shape=jax.ShapeDtypeStruct((_BATCH, 1, 1), jnp.float32),
    )(zt, R.reshape(_BATCH, 1, 1))
    raw = raw2.reshape(_BATCH)

    r2 = R.reshape(1, _BATCH)
    nobs = jnp.asarray(num_obs, jnp.float32).reshape(1, 1)
    mu, sigma2, obj = pl.pallas_call(
        _finish_body,
        in_specs=[
            pl.BlockSpec((1, _BATCH), lambda: (0, 0)),
            pl.BlockSpec((1, _BATCH), lambda: (0, 0)),
            pl.BlockSpec((1, 1), lambda: (0, 0), memory_space=pltpu.SMEM),
        ],
        out_specs=[pl.BlockSpec((1, _BATCH), lambda: (0, 0))] * 3,
        out_shape=[jax.ShapeDtypeStruct((1, _BATCH), jnp.float32)] * 3,
    )(raw.reshape(1, _BATCH), r2, nobs)

    return (raw, mu.reshape(_BATCH), sigma2.reshape(_BATCH),
            obj.reshape(_BATCH))


# speedup vs baseline: 2.6316x; 2.6316x over previous
import functools

import jax
import jax.numpy as jnp
from jax.experimental import pallas as pl
from jax.experimental.pallas import tpu as pltpu

_BATCH = 64
_THRESH2 = 3.0
_ALPHA = 2.0
_BETA = 1.0
_OBS_R = 256
_OBS_S = 512


def _score_body(z_ref, r_ref, out_ref):
    j = pl.program_id(1)
    x = z_ref[0]
    x0 = x[0]
    x1 = x[1]
    x2 = x[2]
    z2 = x0 * x0 + x1 * x1 + x2 * x2
    b_coef = -0.5 / (r_ref[0, 0, 0] * r_ref[0, 0, 0])
    e = jnp.exp(z2 * b_coef)
    scores = jnp.where(z2 < _THRESH2, e, 0.0)
    ssum = jnp.sum(scores)

    @pl.when(j == 0)
    def _init():
        out_ref[0, 0, 0] = ssum

    @pl.when(j != 0)
    def _acc():
        out_ref[0, 0, 0] += ssum


def _finish_body(raw_ref, r_ref, nobs_ref, mu_ref, s2_ref, obj_ref):
    r = r_ref[...]
    a = 1.0 / (r * r)
    b = 0.5 * a
    t2 = _THRESH2
    mu = (1.0 - jnp.exp(-b * t2)) / (4.0 * b)
    mean_s2 = (1.0 - jnp.exp(-2.0 * b * t2)) / (8.0 * b)
    sigma2 = mean_s2 - mu * mu
    n = nobs_ref[0, 0]
    mu = n * mu
    sigma2 = n * sigma2
    mu_ref[...] = mu
    s2_ref[...] = sigma2
    obj_ref[...] = raw_ref[...] - _ALPHA * mu - _BETA * sigma2


@functools.partial(jax.jit, static_argnames=())
def kernel(z, R, num_obs):
    zt = jnp.transpose(z, (0, 3, 1, 2))

    rb = 256
    nj = _OBS_R // rb
    raw2 = pl.pallas_call(
        _score_body,
        grid=(_BATCH, nj),
        in_specs=[
            pl.BlockSpec((1, 3, rb, _OBS_S), lambda b, j: (b, 0, j, 0)),
            pl.BlockSpec((1, 1, 1), lambda b, j: (b, 0, 0),
                         memory_space=pltpu.SMEM),
        ],
        out_specs=pl.BlockSpec((1, 1, 1), lambda b, j: (b, 0, 0),
                               memory_space=pltpu.SMEM),
        out_shape=jax.ShapeDtypeStruct((_BATCH, 1, 1), jnp.float32),
    )(zt, R.reshape(_BATCH, 1, 1))
    raw = raw2.reshape(_BATCH)

    r2 = R.reshape(1, _BATCH)
    nobs = jnp.asarray(num_obs, jnp.float32).reshape(1, 1)
    mu, sigma2, obj = pl.pallas_call(
        _finish_body,
        in_specs=[
            pl.BlockSpec((1, _BATCH), lambda: (0, 0)),
            pl.BlockSpec((1, _BATCH), lambda: (0, 0)),
            pl.BlockSpec((1, 1), lambda: (0, 0), memory_space=pltpu.SMEM),
        ],
        out_specs=[pl.BlockSpec((1, _BATCH), lambda: (0, 0))] * 3,
        out_shape=[jax.ShapeDtypeStruct((1, _BATCH), jnp.float32)] * 3,
    )(raw.reshape(1, _BATCH), r2, nobs)

    return (raw, mu.reshape(_BATCH), sigma2.reshape(_BATCH),
            obj.reshape(_BATCH))
